# Initial kernel scaffold; baseline (speedup 1.0000x reference)
import jax
import jax.numpy as jnp
from jax.experimental import pallas as pl


def kernel(x, edge_index, edge_type, W_n, b_n, W_d, b_d, W_sl, b_sl, W_gat, b_gat):
    raise NotImplementedError("write your pallas kernel here")


# speedup vs baseline: 20.1916x; 20.1916x over previous
import jax
import jax.numpy as jnp
from jax import lax
from jax.experimental import pallas as pl
from jax.experimental.pallas import tpu as pltpu
from jax.experimental.pallas import tpu_sc as plsc

N = 10000
F = 128
E = 320000
TWO_N = 2 * N

NC = 2
NS = 16
NW = NC * NS

NDUM = 64

E_PER_W = E // NW
CH_A = 2000
N_CH_A = E_PER_W // CH_A


def _sc_prep_body(dst_hbm, typ_hbm, degp_out, d0_out, d1_out,
                  dst_v, typ_v, d0_v, d1_v, deg_v):
    c = lax.axis_index("c")
    s = lax.axis_index("s")
    wid = s * NC + c

    def zero_body(j, carry):
        deg_v[pl.ds(pl.multiple_of(j * 16, 16), 16)] = jnp.zeros((16,), jnp.float32)
        return carry

    lax.fori_loop(0, TWO_N // 16, zero_body, 0)

    ones16 = jnp.ones((16,), jnp.float32)
    for c5 in range(N_CH_A):
        base = pl.multiple_of(wid * E_PER_W + c5 * CH_A, 8)
        pltpu.sync_copy(dst_hbm.at[pl.ds(base, CH_A)], dst_v)
        pltpu.sync_copy(typ_hbm.at[pl.ds(base, CH_A)], typ_v)

        def body(j, carry):
            sl = pl.ds(pl.multiple_of(j * 16, 16), 16)
            d16 = dst_v[sl]
            t16 = typ_v[sl]
            dummy = N + (d16 & (NDUM - 1))
            m0 = t16 == 0
            d0_v[sl] = jnp.where(m0, d16, dummy)
            d1_v[sl] = jnp.where(m0, dummy, d16)
            plsc.addupdate_scatter(deg_v, [d16 * 2 + t16], ones16)
            return carry

        lax.fori_loop(0, CH_A // 16, body, 0)
        pltpu.sync_copy(d0_v, d0_out.at[pl.ds(base, CH_A)])
        pltpu.sync_copy(d1_v, d1_out.at[pl.ds(base, CH_A)])

    pltpu.sync_copy(deg_v, degp_out.at[wid])


@jax.jit
def _sc_prep(dst, typ):
    mesh = plsc.VectorSubcoreMesh(core_axis_name="c", subcore_axis_name="s")
    fn = pl.kernel(
        _sc_prep_body,
        mesh=mesh,
        compiler_params=pltpu.CompilerParams(needs_layout_passes=False),
        out_type=[
            jax.ShapeDtypeStruct((NW, TWO_N), jnp.float32),
            jax.ShapeDtypeStruct((E,), jnp.int32),
            jax.ShapeDtypeStruct((E,), jnp.int32),
        ],
        scratch_types=[
            pltpu.VMEM((CH_A,), jnp.int32),
            pltpu.VMEM((CH_A,), jnp.int32),
            pltpu.VMEM((CH_A,), jnp.int32),
            pltpu.VMEM((CH_A,), jnp.int32),
            pltpu.VMEM((TWO_N,), jnp.float32),
        ],
    )
    return fn(dst, typ)


BN = 1000


def _tc_prep_body(degp_ref, x_ref, dinv_ref, y0_ref, y1_ref):
    deg = jnp.sum(degp_ref[...], axis=0) + 1.0
    dinv = lax.rsqrt(deg)
    dinv_ref[...] = dinv
    xb = x_ref[...]
    y0_ref[...] = dinv[:, 0:1] * xb
    y1_ref[...] = dinv[:, 1:2] * xb


@jax.jit
def _tc_prep(degp, x):
    return pl.pallas_call(
        _tc_prep_body,
        grid=(N // BN,),
        in_specs=[
            pl.BlockSpec((NW, BN, 2), lambda i: (0, i, 0)),
            pl.BlockSpec((BN, F), lambda i: (i, 0)),
        ],
        out_specs=[
            pl.BlockSpec((BN, 2), lambda i: (i, 0)),
            pl.BlockSpec((BN, F), lambda i: (i, 0)),
            pl.BlockSpec((BN, F), lambda i: (i, 0)),
        ],
        out_shape=[
            jax.ShapeDtypeStruct((N, 2), jnp.float32),
            jax.ShapeDtypeStruct((N, F), jnp.float32),
            jax.ShapeDtypeStruct((N, F), jnp.float32),
        ],
    )(degp, x)


E_PER_S = E // NS
CH_C = 128
N_CH_C = E_PER_S // CH_C
TAIL_C = E_PER_S - N_CH_C * CH_C
ZROWS = 624
ZTAIL = N - NS * ZROWS
ZCH = 104


def _sc_agg_body(y0, y1, src, d0, d1, qout,
                 six_v, dix_v, rows_v, sixt_v, dixt_v, rowst_v, zbuf, q_sh, sem):
    c = lax.axis_index("c")
    s = lax.axis_index("s")

    z16 = jnp.zeros((16,), jnp.float32)

    def zb_body(r, carry):
        for k in range(F // 16):
            zbuf[r, pl.ds(k * 16, 16)] = z16
        return carry

    lax.fori_loop(0, ZCH, zb_body, 0)
    for kk in range(ZROWS // ZCH):
        zsl = pl.ds(pl.multiple_of(s * ZROWS + kk * ZCH, 8), ZCH)
        pltpu.sync_copy(zbuf, q_sh.at[zsl])

    @pl.when(s < ZTAIL // 8)
    def _():
        tsl = pl.ds(pl.multiple_of(NS * ZROWS + s * 8, 8), 8)
        pltpu.sync_copy(zbuf.at[pl.ds(0, 8)], q_sh.at[tsl])

    @pl.when(s == 2)
    def _():
        dsl = pl.ds(pl.multiple_of(N, 8), NDUM)
        pltpu.sync_copy(zbuf.at[pl.ds(0, NDUM)], q_sh.at[dsl])

    plsc.subcore_barrier()

    ebase = s * E_PER_S

    def edge_loop(ysel, dsel):
        def chunk_body(k, carry):
            b = pl.multiple_of(ebase + k * CH_C, 8)
            pltpu.sync_copy(src.at[pl.ds(b, CH_C)], six_v)
            pltpu.sync_copy(dsel.at[pl.ds(b, CH_C)], dix_v)
            pltpu.async_copy(ysel.at[six_v], rows_v, sem).wait()
            pltpu.sync_copy(rows_v, q_sh.at[dix_v], add=True)
            return carry

        lax.fori_loop(0, N_CH_C, chunk_body, 0)
        bt = pl.multiple_of(ebase + N_CH_C * CH_C, 8)
        pltpu.sync_copy(src.at[pl.ds(bt, TAIL_C)], sixt_v)
        pltpu.sync_copy(dsel.at[pl.ds(bt, TAIL_C)], dixt_v)
        pltpu.async_copy(ysel.at[sixt_v], rowst_v, sem).wait()
        pltpu.sync_copy(rowst_v, q_sh.at[dixt_v], add=True)

    @pl.when(c == 0)
    def _():
        edge_loop(y0, d0)

    @pl.when(c == 1)
    def _():
        edge_loop(y1, d1)

    plsc.subcore_barrier()
    for kk in range(ZROWS // ZCH):
        zsl = pl.ds(pl.multiple_of(s * ZROWS + kk * ZCH, 8), ZCH)
        pltpu.sync_copy(q_sh.at[zsl], zbuf)
        pltpu.sync_copy(zbuf, qout.at[c].at[zsl])

    @pl.when(s < ZTAIL // 8)
    def _():
        tsl = pl.ds(pl.multiple_of(NS * ZROWS + s * 8, 8), 8)
        pltpu.sync_copy(q_sh.at[tsl], zbuf.at[pl.ds(0, 8)])
        pltpu.sync_copy(zbuf.at[pl.ds(0, 8)], qout.at[c].at[tsl])


@jax.jit
def _sc_agg(y0, y1, src, d0, d1):
    mesh = plsc.VectorSubcoreMesh(core_axis_name="c", subcore_axis_name="s")
    fn = pl.kernel(
        _sc_agg_body,
        mesh=mesh,
        compiler_params=pltpu.CompilerParams(needs_layout_passes=False),
        out_type=[jax.ShapeDtypeStruct((NC, N, F), jnp.float32)],
        scratch_types=[
            pltpu.VMEM((CH_C,), jnp.int32),
            pltpu.VMEM((CH_C,), jnp.int32),
            pltpu.VMEM((CH_C, F), jnp.float32),
            pltpu.VMEM((TAIL_C,), jnp.int32),
            pltpu.VMEM((TAIL_C,), jnp.int32),
            pltpu.VMEM((TAIL_C, F), jnp.float32),
            pltpu.VMEM((ZCH, F), jnp.float32),
            pltpu.VMEM_SHARED((N + NDUM, F), jnp.float32),
            pltpu.SemaphoreType.DMA,
        ],
    )
    return fn(y0, y1, src, d0, d1)


def _tc_final_body(x_ref, qn_ref, qd_ref, dinv_ref,
                   wn, bn, wd, bd, wsl, bsl, wg, bg, wdr, bdr, out_ref):
    xb = x_ref[...]
    d0 = dinv_ref[:, 0:1]
    d1 = dinv_ref[:, 1:2]
    pre_n = d0 * (qn_ref[0] + d0 * xb)
    pre_d = d1 * (qd_ref[0] + d1 * xb)

    def mm(a, w):
        return jnp.dot(a, w[...], preferred_element_type=jnp.float32)

    xn = mm(pre_n, wn) + bn[...]
    xd = mm(pre_d, wd) + bd[...]
    xx = mm(xb, wsl) + bsl[...]
    wgr = wg[...]
    z = (jnp.dot(xx, wgr[0:F, :], preferred_element_type=jnp.float32)
         + jnp.dot(xn, wgr[F:2 * F, :], preferred_element_type=jnp.float32)
         + jnp.dot(xd, wgr[2 * F:3 * F, :], preferred_element_type=jnp.float32)
         + bg[...])
    m = jnp.max(z, axis=-1, keepdims=True)
    ez = jnp.exp(z - m)
    sm = ez / jnp.sum(ez, axis=-1, keepdims=True)
    rr = lax.broadcasted_iota(jnp.int32, (F, F), 0)
    cc = lax.broadcasted_iota(jnp.int32, (F, F), 1)
    tri = (rr <= cc).astype(jnp.float32)
    gat = jnp.dot(sm, tri, preferred_element_type=jnp.float32)
    xdr = mm(pre_d, wdr) + bdr[...]
    out_ref[...] = xdr * gat + xx + xn


@jax.jit
def _tc_final(x, q2, dinv, wn, bn, wd, bd, wsl, bsl, wg, bg, wdr, bdr):
    wspec = lambda shape: pl.BlockSpec(shape, lambda i: tuple(0 for _ in shape))
    return pl.pallas_call(
        _tc_final_body,
        grid=(N // BN,),
        in_specs=[
            pl.BlockSpec((BN, F), lambda i: (i, 0)),
            pl.BlockSpec((1, BN, F), lambda i: (0, i, 0)),
            pl.BlockSpec((1, BN, F), lambda i: (1, i, 0)),
            pl.BlockSpec((BN, 2), lambda i: (i, 0)),
            wspec((F, F)), wspec((1, F)),
            wspec((F, F)), wspec((1, F)),
            wspec((F, F)), wspec((1, F)),
            wspec((3 * F, F)), wspec((1, F)),
            wspec((F, F)), wspec((1, F)),
        ],
        out_specs=pl.BlockSpec((BN, F), lambda i: (i, 0)),
        out_shape=jax.ShapeDtypeStruct((N, F), jnp.float32),
    )(x, q2, q2, dinv, wn, bn, wd, bd, wsl, bsl, wg, bg, wdr, bdr)


def kernel(x, edge_index, edge_type, W_n, b_n, W_d, b_d, W_sl, b_sl, W_gat, b_gat):
    src = edge_index[0].astype(jnp.int32)
    dst = edge_index[1].astype(jnp.int32)
    typ = edge_type.astype(jnp.int32)

    degp, d0, d1 = _sc_prep(dst, typ)
    dinv, y0, y1 = _tc_prep(degp.reshape(NW, N, 2), x)
    (q2,) = _sc_agg(y0, y1, src, d0, d1)
    out = _tc_final(
        x, q2, dinv,
        W_n, b_n.reshape(1, F), W_d, b_d.reshape(1, F),
        W_sl, b_sl.reshape(1, F), W_gat, b_gat.reshape(1, F),
        W_d[:, ::-1], b_d[::-1].reshape(1, F),
    )
    return out


# speedup vs baseline: 27.7430x; 1.3740x over previous
import jax
import jax.numpy as jnp
from jax import lax
from jax.experimental import pallas as pl
from jax.experimental.pallas import tpu as pltpu
from jax.experimental.pallas import tpu_sc as plsc

N = 10000
F = 128
E = 320000
TWO_N = 2 * N

NC = 2
NS = 16
NW = NC * NS

NDUM = 64

E_PER_W = E // NW
CH_A = 2000
N_CH_A = E_PER_W // CH_A


def _sc_prep_body(dst_hbm, typ_hbm, degp_out, d0_out, d1_out,
                  dst_v, typ_v, d0_v, d1_v, deg_v):
    c = lax.axis_index("c")
    s = lax.axis_index("s")
    wid = s * NC + c

    def zero_body(j, carry):
        deg_v[pl.ds(pl.multiple_of(j * 16, 16), 16)] = jnp.zeros((16,), jnp.float32)
        return carry

    lax.fori_loop(0, TWO_N // 16, zero_body, 0)

    ones16 = jnp.ones((16,), jnp.float32)
    for c5 in range(N_CH_A):
        base = pl.multiple_of(wid * E_PER_W + c5 * CH_A, 8)
        pltpu.sync_copy(dst_hbm.at[pl.ds(base, CH_A)], dst_v)
        pltpu.sync_copy(typ_hbm.at[pl.ds(base, CH_A)], typ_v)

        def body(j, carry):
            sl = pl.ds(pl.multiple_of(j * 16, 16), 16)
            d16 = dst_v[sl]
            t16 = typ_v[sl]
            dummy = N + (d16 & (NDUM - 1))
            m0 = t16 == 0
            d0_v[sl] = jnp.where(m0, d16, dummy)
            d1_v[sl] = jnp.where(m0, dummy, d16)
            plsc.addupdate_scatter(deg_v, [d16 * 2 + t16], ones16)
            return carry

        lax.fori_loop(0, CH_A // 16, body, 0)
        pltpu.sync_copy(d0_v, d0_out.at[pl.ds(base, CH_A)])
        pltpu.sync_copy(d1_v, d1_out.at[pl.ds(base, CH_A)])

    pltpu.sync_copy(deg_v, degp_out.at[wid])


@jax.jit
def _sc_prep(dst, typ):
    mesh = plsc.VectorSubcoreMesh(core_axis_name="c", subcore_axis_name="s")
    fn = pl.kernel(
        _sc_prep_body,
        mesh=mesh,
        compiler_params=pltpu.CompilerParams(needs_layout_passes=False),
        out_type=[
            jax.ShapeDtypeStruct((NW, TWO_N), jnp.float32),
            jax.ShapeDtypeStruct((E,), jnp.int32),
            jax.ShapeDtypeStruct((E,), jnp.int32),
        ],
        scratch_types=[
            pltpu.VMEM((CH_A,), jnp.int32),
            pltpu.VMEM((CH_A,), jnp.int32),
            pltpu.VMEM((CH_A,), jnp.int32),
            pltpu.VMEM((CH_A,), jnp.int32),
            pltpu.VMEM((TWO_N,), jnp.float32),
        ],
    )
    return fn(dst, typ)


BN = 1000


def _tc_prep_body(degp_ref, x_ref, dinv_ref, y0_ref, y1_ref):
    deg = jnp.sum(degp_ref[...], axis=0) + 1.0
    dinv = lax.rsqrt(deg)
    dinv_ref[...] = dinv
    xb = x_ref[...]
    y0_ref[...] = dinv[:, 0:1] * xb
    y1_ref[...] = dinv[:, 1:2] * xb


@jax.jit
def _tc_prep(degp, x):
    return pl.pallas_call(
        _tc_prep_body,
        grid=(N // BN,),
        in_specs=[
            pl.BlockSpec((NW, BN, 2), lambda i: (0, i, 0)),
            pl.BlockSpec((BN, F), lambda i: (i, 0)),
        ],
        out_specs=[
            pl.BlockSpec((BN, 2), lambda i: (i, 0)),
            pl.BlockSpec((BN, F), lambda i: (i, 0)),
            pl.BlockSpec((BN, F), lambda i: (i, 0)),
        ],
        out_shape=[
            jax.ShapeDtypeStruct((N, 2), jnp.float32),
            jax.ShapeDtypeStruct((N, F), jnp.float32),
            jax.ShapeDtypeStruct((N, F), jnp.float32),
        ],
    )(degp, x)


E_PER_S = E // NS
CH_C = 128
N_CH_C = E_PER_S // CH_C
TAIL_C = E_PER_S - N_CH_C * CH_C
ZROWS = 624
ZTAIL = N - NS * ZROWS
ZCH = 48


def _sc_agg_body(y0, y1, src, d0, d1, qout,
                 six_v0, six_v1, dix_v0, dix_v1, rows_v0, rows_v1,
                 sixt_v, dixt_v, rowst_v, zbuf, q_sh,
                 semg0, semg1, sems0, sems1, sem):
    c = lax.axis_index("c")
    s = lax.axis_index("s")
    sixs = (six_v0, six_v1)
    dixs = (dix_v0, dix_v1)
    rows = (rows_v0, rows_v1)
    semg = (semg0, semg1)
    sems = (sems0, sems1)

    z16 = jnp.zeros((16,), jnp.float32)

    def zb_body(r, carry):
        for k in range(F // 16):
            zbuf[r, pl.ds(k * 16, 16)] = z16
        return carry

    lax.fori_loop(0, ZCH, zb_body, 0)
    for kk in range(ZROWS // ZCH):
        zsl = pl.ds(pl.multiple_of(s * ZROWS + kk * ZCH, 8), ZCH)
        pltpu.sync_copy(zbuf, q_sh.at[zsl])

    @pl.when(s < ZTAIL // 8)
    def _():
        tsl = pl.ds(pl.multiple_of(NS * ZROWS + s * 8, 8), 8)
        pltpu.sync_copy(zbuf.at[pl.ds(0, 8)], q_sh.at[tsl])

    @pl.when(s == 2)
    def _():
        dsl = pl.ds(pl.multiple_of(N, 8), NDUM)
        pltpu.sync_copy(zbuf.at[pl.ds(0, NDUM)], q_sh.at[dsl])

    plsc.subcore_barrier()

    ebase = s * E_PER_S

    def edge_loop(ysel, dsel):
        b0 = pl.multiple_of(ebase, 8)
        pltpu.sync_copy(src.at[pl.ds(b0, CH_C)], sixs[0])
        pltpu.sync_copy(dsel.at[pl.ds(b0, CH_C)], dixs[0])
        pltpu.async_copy(ysel.at[sixs[0]], rows[0], semg[0])

        def pair_body(k2, carry):
            for b in (0, 1):
                o = 1 - b
                k = k2 * 2 + b

                @pl.when(k > 0)
                def _():
                    pltpu.make_async_copy(rows[o], q_sh.at[dixs[o]], sems[o]).wait()

                @pl.when(k < N_CH_C - 1)
                def _():
                    bnx = pl.multiple_of(ebase + (k + 1) * CH_C, 8)
                    pltpu.sync_copy(src.at[pl.ds(bnx, CH_C)], sixs[o])
                    pltpu.sync_copy(dsel.at[pl.ds(bnx, CH_C)], dixs[o])

                pltpu.make_async_copy(ysel.at[sixs[b]], rows[b], semg[b]).wait()

                @pl.when(k < N_CH_C - 1)
                def _():
                    pltpu.async_copy(ysel.at[sixs[o]], rows[o], semg[o])

                pltpu.async_copy(rows[b], q_sh.at[dixs[b]], sems[b], add=True)
            return carry

        lax.fori_loop(0, N_CH_C // 2, pair_body, 0)
        pltpu.make_async_copy(rows[1], q_sh.at[dixs[1]], sems[1]).wait()

        bt = pl.multiple_of(ebase + N_CH_C * CH_C, 8)
        pltpu.sync_copy(src.at[pl.ds(bt, TAIL_C)], sixt_v)
        pltpu.sync_copy(dsel.at[pl.ds(bt, TAIL_C)], dixt_v)
        pltpu.async_copy(ysel.at[sixt_v], rowst_v, sem).wait()
        pltpu.sync_copy(rowst_v, q_sh.at[dixt_v], add=True)

    @pl.when(c == 0)
    def _():
        edge_loop(y0, d0)

    @pl.when(c == 1)
    def _():
        edge_loop(y1, d1)

    plsc.subcore_barrier()
    for kk in range(ZROWS // ZCH):
        zsl = pl.ds(pl.multiple_of(s * ZROWS + kk * ZCH, 8), ZCH)
        pltpu.sync_copy(q_sh.at[zsl], zbuf)
        pltpu.sync_copy(zbuf, qout.at[c].at[zsl])

    @pl.when(s < ZTAIL // 8)
    def _():
        tsl = pl.ds(pl.multiple_of(NS * ZROWS + s * 8, 8), 8)
        pltpu.sync_copy(q_sh.at[tsl], zbuf.at[pl.ds(0, 8)])
        pltpu.sync_copy(zbuf.at[pl.ds(0, 8)], qout.at[c].at[tsl])


@jax.jit
def _sc_agg(y0, y1, src, d0, d1):
    mesh = plsc.VectorSubcoreMesh(core_axis_name="c", subcore_axis_name="s")
    fn = pl.kernel(
        _sc_agg_body,
        mesh=mesh,
        compiler_params=pltpu.CompilerParams(needs_layout_passes=False),
        out_type=[jax.ShapeDtypeStruct((NC, N, F), jnp.float32)],
        scratch_types=[
            pltpu.VMEM((CH_C,), jnp.int32),
            pltpu.VMEM((CH_C,), jnp.int32),
            pltpu.VMEM((CH_C,), jnp.int32),
            pltpu.VMEM((CH_C,), jnp.int32),
            pltpu.VMEM((CH_C, F), jnp.float32),
            pltpu.VMEM((CH_C, F), jnp.float32),
            pltpu.VMEM((TAIL_C,), jnp.int32),
            pltpu.VMEM((TAIL_C,), jnp.int32),
            pltpu.VMEM((TAIL_C, F), jnp.float32),
            pltpu.VMEM((ZCH, F), jnp.float32),
            pltpu.VMEM_SHARED((N + NDUM, F), jnp.float32),
            pltpu.SemaphoreType.DMA,
            pltpu.SemaphoreType.DMA,
            pltpu.SemaphoreType.DMA,
            pltpu.SemaphoreType.DMA,
            pltpu.SemaphoreType.DMA,
        ],
    )
    return fn(y0, y1, src, d0, d1)


def _tc_final_body(x_ref, qn_ref, qd_ref, dinv_ref,
                   wn, bn, wd, bd, wsl, bsl, wg, bg, wdr, bdr, out_ref):
    xb = x_ref[...]
    d0 = dinv_ref[:, 0:1]
    d1 = dinv_ref[:, 1:2]
    pre_n = d0 * (qn_ref[0] + d0 * xb)
    pre_d = d1 * (qd_ref[0] + d1 * xb)

    def mm(a, w):
        return jnp.dot(a, w[...], preferred_element_type=jnp.float32)

    xn = mm(pre_n, wn) + bn[...]
    xd = mm(pre_d, wd) + bd[...]
    xx = mm(xb, wsl) + bsl[...]
    wgr = wg[...]
    z = (jnp.dot(xx, wgr[0:F, :], preferred_element_type=jnp.float32)
         + jnp.dot(xn, wgr[F:2 * F, :], preferred_element_type=jnp.float32)
         + jnp.dot(xd, wgr[2 * F:3 * F, :], preferred_element_type=jnp.float32)
         + bg[...])
    m = jnp.max(z, axis=-1, keepdims=True)
    ez = jnp.exp(z - m)
    sm = ez / jnp.sum(ez, axis=-1, keepdims=True)
    rr = lax.broadcasted_iota(jnp.int32, (F, F), 0)
    cc = lax.broadcasted_iota(jnp.int32, (F, F), 1)
    tri = (rr <= cc).astype(jnp.float32)
    gat = jnp.dot(sm, tri, preferred_element_type=jnp.float32)
    xdr = mm(pre_d, wdr) + bdr[...]
    out_ref[...] = xdr * gat + xx + xn


@jax.jit
def _tc_final(x, q2, dinv, wn, bn, wd, bd, wsl, bsl, wg, bg, wdr, bdr):
    wspec = lambda shape: pl.BlockSpec(shape, lambda i: tuple(0 for _ in shape))
    return pl.pallas_call(
        _tc_final_body,
        grid=(N // BN,),
        in_specs=[
            pl.BlockSpec((BN, F), lambda i: (i, 0)),
            pl.BlockSpec((1, BN, F), lambda i: (0, i, 0)),
            pl.BlockSpec((1, BN, F), lambda i: (1, i, 0)),
            pl.BlockSpec((BN, 2), lambda i: (i, 0)),
            wspec((F, F)), wspec((1, F)),
            wspec((F, F)), wspec((1, F)),
            wspec((F, F)), wspec((1, F)),
            wspec((3 * F, F)), wspec((1, F)),
            wspec((F, F)), wspec((1, F)),
        ],
        out_specs=pl.BlockSpec((BN, F), lambda i: (i, 0)),
        out_shape=jax.ShapeDtypeStruct((N, F), jnp.float32),
    )(x, q2, q2, dinv, wn, bn, wd, bd, wsl, bsl, wg, bg, wdr, bdr)


def kernel(x, edge_index, edge_type, W_n, b_n, W_d, b_d, W_sl, b_sl, W_gat, b_gat):
    src = edge_index[0].astype(jnp.int32)
    dst = edge_index[1].astype(jnp.int32)
    typ = edge_type.astype(jnp.int32)

    degp, d0, d1 = _sc_prep(dst, typ)
    dinv, y0, y1 = _tc_prep(degp.reshape(NW, N, 2), x)
    (q2,) = _sc_agg(y0, y1, src, d0, d1)
    out = _tc_final(
        x, q2, dinv,
        W_n, b_n.reshape(1, F), W_d, b_d.reshape(1, F),
        W_sl, b_sl.reshape(1, F), W_gat, b_gat.reshape(1, F),
        W_d[:, ::-1], b_d[::-1].reshape(1, F),
    )
    return out
